# baseline (speedup 1.0000x reference)
import functools

import jax
import jax.numpy as jnp
from jax import lax
from jax.experimental import pallas as pl
from jax.experimental.pallas import tpu as pltpu
from jax.experimental.pallas import tpu_sc as plsc

H = 128
N = 10000
E = 320000
G = 64
OUT = 5

NC = 2
NS = 16
L = 16
NW = NC * NS
EPW = E // NW
CK = 80
NCHUNK = EPW // CK
RPT = 624
TAIL = N - NS * RPT

_NV = H // L

_GD = lax.GatherDimensionNumbers(
    offset_dims=(), collapsed_slice_dims=(0,), start_index_map=(0,))


def _bcast_lane(vec, j):
    idx = jnp.full((L, 1), j, jnp.int32)
    return lax.gather(vec, idx, _GD, (1,),
                      mode=lax.GatherScatterMode.PROMISE_IN_BOUNDS)


def _sc_gather_nw(src_h, nw_h, out_h, srcv, nwg, sem):
    c = lax.axis_index("c")
    s = lax.axis_index("s")
    wid = c * NS + s

    def chunk(ci, carry):
        base = wid * EPW + ci * CK
        pltpu.sync_copy(src_h.at[pl.ds(base, CK)], srcv)
        pltpu.async_copy(nw_h.at[srcv], nwg, sem).wait()
        pltpu.sync_copy(nwg, out_h.at[pl.ds(base, CK)])
        return carry

    lax.fori_loop(0, NCHUNK, chunk, 0)


def _sc_scatter(dst_h, msg_h, zero_h, out_h, dstv, mbuf, acc):
    c = lax.axis_index("c")
    s = lax.axis_index("s")
    wid = c * NS + s
    pltpu.sync_copy(zero_h.at[pl.ds(0, RPT)], acc.at[pl.ds(s * RPT, RPT)])

    @pl.when(s == 0)
    def _zero_tail():
        pltpu.sync_copy(zero_h.at[pl.ds(0, TAIL)], acc.at[pl.ds(NS * RPT, TAIL)])

    plsc.subcore_barrier()

    def chunk(ci, carry):
        base = wid * EPW + ci * CK
        pltpu.sync_copy(dst_h.at[pl.ds(base, CK)], dstv)
        pltpu.sync_copy(msg_h.at[pl.ds(base, CK)], mbuf)
        pltpu.sync_copy(mbuf, acc.at[dstv], add=True)
        return carry

    lax.fori_loop(0, NCHUNK, chunk, 0)
    plsc.subcore_barrier()
    pltpu.sync_copy(acc.at[pl.ds(s * RPT, RPT)],
                    out_h.at[c, pl.ds(s * RPT, RPT)])

    @pl.when(s == 0)
    def _write_tail():
        pltpu.sync_copy(acc.at[pl.ds(NS * RPT, TAIL)],
                        out_h.at[c, pl.ds(NS * RPT, TAIL)])


def _sc_layer2(src_h, dst_h, ew_h, x_h, coef_h, zero_h, out_h,
               srcv, dstv, ewv, rows, coefv, acc, sem):
    c = lax.axis_index("c")
    s = lax.axis_index("s")
    wid = c * NS + s
    pltpu.sync_copy(coef_h, coefv)
    pltpu.sync_copy(zero_h.at[pl.ds(0, RPT)], acc.at[pl.ds(s * RPT, RPT)])

    @pl.when(s == 0)
    def _zero_tail():
        pltpu.sync_copy(zero_h.at[pl.ds(0, TAIL)], acc.at[pl.ds(NS * RPT, TAIL)])

    plsc.subcore_barrier()
    we = [coefv[0, pl.ds(L * v, L)] for v in range(_NV)]
    eb = [coefv[1, pl.ds(L * v, L)] for v in range(_NV)]

    def chunk(ci, carry):
        base = wid * EPW + ci * CK
        pltpu.sync_copy(src_h.at[pl.ds(base, CK)], srcv)
        pltpu.sync_copy(dst_h.at[pl.ds(base, CK)], dstv)
        pltpu.sync_copy(ew_h.at[pl.ds(base, CK)], ewv)
        pltpu.async_copy(x_h.at[srcv], rows, sem).wait()

        def group(q, cr):
            b16 = q * L
            ew16 = ewv[pl.ds(b16, L)]
            for j in range(L):
                ewb = _bcast_lane(ew16, j)
                for v in range(_NV):
                    sl = pl.ds(L * v, L)
                    t = rows[b16 + j, sl] + (ewb * we[v] + eb[v])
                    rows[b16 + j, sl] = jnp.maximum(t, 0.0)
            return cr

        lax.fori_loop(0, CK // L, group, 0)
        pltpu.sync_copy(rows, acc.at[dstv], add=True)
        return carry

    lax.fori_loop(0, NCHUNK, chunk, 0)
    plsc.subcore_barrier()
    pltpu.sync_copy(acc.at[pl.ds(s * RPT, RPT)],
                    out_h.at[c, pl.ds(s * RPT, RPT)])

    @pl.when(s == 0)
    def _write_tail():
        pltpu.sync_copy(acc.at[pl.ds(NS * RPT, TAIL)],
                        out_h.at[c, pl.ds(NS * RPT, TAIL)])


@functools.cache
def _build_sc_kernels():
    mesh = plsc.VectorSubcoreMesh(
        core_axis_name="c", subcore_axis_name="s",
        num_cores=NC, num_subcores=NS)
    sc_gnw = pl.kernel(
        _sc_gather_nw,
        out_type=jax.ShapeDtypeStruct((E,), jnp.float32),
        mesh=mesh,
        scratch_types=[
            pltpu.VMEM((CK,), jnp.int32),
            pltpu.VMEM((CK,), jnp.float32),
            pltpu.SemaphoreType.DMA,
        ],
    )
    sc_sct = pl.kernel(
        _sc_scatter,
        out_type=jax.ShapeDtypeStruct((NC, N, H), jnp.float32),
        mesh=mesh,
        scratch_types=[
            pltpu.VMEM((CK,), jnp.int32),
            pltpu.VMEM((CK, H), jnp.float32),
            pltpu.VMEM_SHARED((N, H), jnp.float32),
        ],
    )
    sc_l2 = pl.kernel(
        _sc_layer2,
        out_type=jax.ShapeDtypeStruct((NC, N, H), jnp.float32),
        mesh=mesh,
        scratch_types=[
            pltpu.VMEM((CK,), jnp.int32),
            pltpu.VMEM((CK,), jnp.int32),
            pltpu.VMEM((CK,), jnp.float32),
            pltpu.VMEM((CK, H), jnp.float32),
            pltpu.VMEM((2, H), jnp.float32),
            pltpu.VMEM_SHARED((N, H), jnp.float32),
            pltpu.SemaphoreType.DMA,
        ],
    )
    return sc_gnw, sc_sct, sc_l2


BE = 8000


def _tc_msg1(a_ref, b_ref, w0_ref, we_ref, cc_ref, out_ref):
    t = a_ref[...] * w0_ref[...] + (b_ref[...] * we_ref[...] + cc_ref[...])
    out_ref[...] = jnp.maximum(t, 0.0)


_msg1 = pl.pallas_call(
    _tc_msg1,
    grid=(E // BE,),
    in_specs=[
        pl.BlockSpec((BE, 1), lambda i: (i, 0)),
        pl.BlockSpec((BE, 1), lambda i: (i, 0)),
        pl.BlockSpec((1, H), lambda i: (0, 0)),
        pl.BlockSpec((1, H), lambda i: (0, 0)),
        pl.BlockSpec((1, H), lambda i: (0, 0)),
    ],
    out_specs=pl.BlockSpec((BE, H), lambda i: (i, 0)),
    out_shape=jax.ShapeDtypeStruct((E, H), jnp.float32),
)


def _bn_rows(t, g, be):
    m = jnp.mean(t, axis=0, keepdims=True)
    v = jnp.mean((t - m) ** 2, axis=0, keepdims=True)
    return g * (t - m) * lax.rsqrt(v + 1e-5) + be


def _tc_dense1(nw_ref, agg_ref, w0_ref, b0_ref, w1_ref, b1_ref,
               g1_ref, be1_ref, w2_ref, b2_ref, out_ref):
    x = nw_ref[...] * w0_ref[...] + b0_ref[...]
    h = x + agg_ref[0] + agg_ref[1]
    t = jnp.dot(h, w1_ref[...], preferred_element_type=jnp.float32) + b1_ref[...]
    t = jnp.maximum(t, 0.0)
    tn = _bn_rows(t, g1_ref[...], be1_ref[...])
    y = jnp.dot(tn, w2_ref[...], preferred_element_type=jnp.float32) + b2_ref[...]
    out_ref[...] = jnp.maximum(y, 0.0)


def _tc_dense2(x_ref, agg_ref, batch_ref, w1_ref, b1_ref, g1_ref, be1_ref,
               w2_ref, b2_ref, mw1_ref, mb1_ref, mg_ref, mbe_ref,
               mw2_ref, mb2_ref, out_ref):
    h = x_ref[...] + agg_ref[0] + agg_ref[1]
    t = jnp.dot(h, w1_ref[...], preferred_element_type=jnp.float32) + b1_ref[...]
    t = jnp.maximum(t, 0.0)
    tn = _bn_rows(t, g1_ref[...], be1_ref[...])
    y = jnp.dot(tn, w2_ref[...], preferred_element_type=jnp.float32) + b2_ref[...]
    x3 = jnp.maximum(y, 0.0)
    oh = (batch_ref[...] == lax.broadcasted_iota(jnp.int32, (1, G), 1))
    pooled = lax.dot_general(oh.astype(jnp.float32), x3,
                             (((0,), (0,)), ((), ())),
                             preferred_element_type=jnp.float32)
    hh = jnp.dot(pooled, mw1_ref[...], preferred_element_type=jnp.float32)
    hh = jnp.maximum(hh + mb1_ref[...], 0.0)
    hn = _bn_rows(hh, mg_ref[...], mbe_ref[...])
    logits = jnp.dot(hn, mw2_ref[...], preferred_element_type=jnp.float32)
    logits = logits + mb2_ref[...]
    z = logits - jnp.max(logits, axis=1, keepdims=True)
    out_ref[...] = z - jnp.log(jnp.sum(jnp.exp(z), axis=1, keepdims=True))


_tc1 = pl.pallas_call(
    _tc_dense1, out_shape=jax.ShapeDtypeStruct((N, H), jnp.float32))
_tc2 = pl.pallas_call(
    _tc_dense2, out_shape=jax.ShapeDtypeStruct((G, OUT), jnp.float32))


def kernel(edge_index, edge_weight, node_weight, batch,
           node_W, node_b, edge_W, edge_b,
           c1_W1, c1_b1, c1_g, c1_be, c1_W2, c1_b2,
           c2_W1, c2_b1, c2_g, c2_be, c2_W2, c2_b2,
           m_W1, m_b1, m_g, m_be, m_W2, m_b2):
    sc_gnw, sc_sct, sc_l2 = _build_sc_kernels()
    src = edge_index[0]
    dst = edge_index[1]
    coef2 = jnp.stack([edge_W[0], edge_b])
    zeros = jnp.zeros((RPT, H), jnp.float32)

    a = sc_gnw(src, node_weight)
    msg1 = _msg1(a.reshape(E, 1), edge_weight.reshape(E, 1),
                 node_W, edge_W, (node_b + edge_b).reshape(1, H))
    agg1 = sc_sct(dst, msg1, zeros)
    x2 = _tc1(node_weight.reshape(N, 1), agg1,
              node_W, node_b.reshape(1, H),
              c1_W1, c1_b1.reshape(1, H), c1_g.reshape(1, H),
              c1_be.reshape(1, H), c1_W2, c1_b2.reshape(1, H))
    agg2 = sc_l2(src, dst, edge_weight, x2, coef2, zeros)
    out = _tc2(x2, agg2, batch.reshape(N, 1),
               c2_W1, c2_b1.reshape(1, H), c2_g.reshape(1, H),
               c2_be.reshape(1, H), c2_W2, c2_b2.reshape(1, H),
               m_W1, m_b1.reshape(1, H), m_g.reshape(1, H),
               m_be.reshape(1, H), m_W2, m_b2.reshape(1, OUT))
    return out


# speedup vs baseline: 1.9053x; 1.9053x over previous
import functools

import jax
import jax.numpy as jnp
from jax import lax
from jax.experimental import pallas as pl
from jax.experimental.pallas import tpu as pltpu
from jax.experimental.pallas import tpu_sc as plsc

H = 128
N = 10000
E = 320000
G = 64
OUT = 5

NC = 2
NS = 16
L = 16
NW = NC * NS
EPW = E // NW
CK = 80
NCHUNK = EPW // CK
NBUF = 5
NSUP = NCHUNK // NBUF
NBUF2 = 4
NSUP2 = NCHUNK // NBUF2
NTAIL2 = NCHUNK - NSUP2 * NBUF2
RPT = 624
TAIL = N - NS * RPT

_NV = H // L

_GD = lax.GatherDimensionNumbers(
    offset_dims=(), collapsed_slice_dims=(0,), start_index_map=(0,))


def _bcast_lane(vec, j):
    idx = jnp.full((L, 1), j, jnp.int32)
    return lax.gather(vec, idx, _GD, (1,),
                      mode=lax.GatherScatterMode.PROMISE_IN_BOUNDS)


def _zero_acc(s, zero_h, acc):
    pltpu.sync_copy(zero_h.at[pl.ds(0, RPT)], acc.at[pl.ds(s * RPT, RPT)])

    @pl.when(s == 0)
    def _zero_tail():
        pltpu.sync_copy(zero_h.at[pl.ds(0, TAIL)], acc.at[pl.ds(NS * RPT, TAIL)])


def _write_out(c, s, acc, out_h):
    pltpu.sync_copy(acc.at[pl.ds(s * RPT, RPT)],
                    out_h.at[c, pl.ds(s * RPT, RPT)])

    @pl.when(s == 0)
    def _write_tail():
        pltpu.sync_copy(acc.at[pl.ds(NS * RPT, TAIL)],
                        out_h.at[c, pl.ds(NS * RPT, TAIL)])


def _sc_layer1(src_h, dst_h, ew_h, nw_h, coef_h, zero_h, out_h, *scr):
    srcv = scr[0:NBUF]
    dstv = scr[NBUF:2 * NBUF]
    ewv = scr[2 * NBUF:3 * NBUF]
    nwg = scr[3 * NBUF:4 * NBUF]
    mbuf, coefv, acc, semL, semG = scr[4 * NBUF:]
    c = lax.axis_index("c")
    s = lax.axis_index("s")
    wid = c * NS + s
    pltpu.sync_copy(coef_h, coefv)
    _zero_acc(s, zero_h, acc)
    plsc.subcore_barrier()
    w0 = [coefv[0, pl.ds(L * v, L)] for v in range(_NV)]
    we = [coefv[1, pl.ds(L * v, L)] for v in range(_NV)]
    cc = [coefv[2, pl.ds(L * v, L)] for v in range(_NV)]

    def superiter(g, carry):
        base0 = wid * EPW + g * NBUF * CK
        lh = []
        for b in range(NBUF):
            base = base0 + b * CK
            lh.append((
                pltpu.async_copy(src_h.at[pl.ds(base, CK)], srcv[b], semL.at[b]),
                pltpu.async_copy(dst_h.at[pl.ds(base, CK)], dstv[b], semL.at[b]),
                pltpu.async_copy(ew_h.at[pl.ds(base, CK)], ewv[b], semL.at[b]),
            ))
        gh = []
        for b in range(NBUF):
            for h in lh[b]:
                h.wait()
            gh.append(pltpu.async_copy(nw_h.at[srcv[b]], nwg[b], semG.at[b]))
        for b in range(NBUF):
            gh[b].wait()

            def group(q, cr, b=b):
                b16 = q * L
                nw16 = nwg[b][pl.ds(b16, L)]
                ew16 = ewv[b][pl.ds(b16, L)]
                for j in range(L):
                    nwb = _bcast_lane(nw16, j)
                    ewb = _bcast_lane(ew16, j)
                    for v in range(_NV):
                        t = nwb * w0[v] + (ewb * we[v] + cc[v])
                        mbuf[b16 + j, pl.ds(L * v, L)] = jnp.maximum(t, 0.0)
                return cr

            lax.fori_loop(0, CK // L, group, 0)
            pltpu.sync_copy(mbuf, acc.at[dstv[b]], add=True)
        return carry

    lax.fori_loop(0, NSUP, superiter, 0)
    plsc.subcore_barrier()
    _write_out(c, s, acc, out_h)


def _sc_layer2(src_h, dst_h, ew_h, x_h, coef_h, zero_h, out_h, *scr):
    srcv = scr[0:NBUF2]
    dstv = scr[NBUF2:2 * NBUF2]
    ewv = scr[2 * NBUF2:3 * NBUF2]
    rows = scr[3 * NBUF2:4 * NBUF2]
    coefv, acc, semL, semG = scr[4 * NBUF2:]
    c = lax.axis_index("c")
    s = lax.axis_index("s")
    wid = c * NS + s
    pltpu.sync_copy(coef_h, coefv)
    _zero_acc(s, zero_h, acc)
    plsc.subcore_barrier()
    we = [coefv[0, pl.ds(L * v, L)] for v in range(_NV)]
    eb = [coefv[1, pl.ds(L * v, L)] for v in range(_NV)]

    def chunk_compute(b):
        def group(q, cr):
            b16 = q * L
            ew16 = ewv[b][pl.ds(b16, L)]
            for j in range(L):
                ewb = _bcast_lane(ew16, j)
                for v in range(_NV):
                    sl = pl.ds(L * v, L)
                    t = rows[b][b16 + j, sl] + (ewb * we[v] + eb[v])
                    rows[b][b16 + j, sl] = jnp.maximum(t, 0.0)
            return cr

        lax.fori_loop(0, CK // L, group, 0)
        pltpu.sync_copy(rows[b], acc.at[dstv[b]], add=True)

    def superiter(g, carry):
        base0 = wid * EPW + g * NBUF2 * CK
        lh = []
        for b in range(NBUF2):
            base = base0 + b * CK
            lh.append((
                pltpu.async_copy(src_h.at[pl.ds(base, CK)], srcv[b], semL.at[b]),
                pltpu.async_copy(dst_h.at[pl.ds(base, CK)], dstv[b], semL.at[b]),
                pltpu.async_copy(ew_h.at[pl.ds(base, CK)], ewv[b], semL.at[b]),
            ))
        gh = []
        for b in range(NBUF2):
            for h in lh[b]:
                h.wait()
            gh.append(pltpu.async_copy(x_h.at[srcv[b]], rows[b], semG.at[b]))
        for b in range(NBUF2):
            gh[b].wait()
            chunk_compute(b)
        return carry

    lax.fori_loop(0, NSUP2, superiter, 0)
    for t in range(NTAIL2):
        base = wid * EPW + (NSUP2 * NBUF2 + t) * CK
        pltpu.sync_copy(src_h.at[pl.ds(base, CK)], srcv[0])
        pltpu.sync_copy(dst_h.at[pl.ds(base, CK)], dstv[0])
        pltpu.sync_copy(ew_h.at[pl.ds(base, CK)], ewv[0])
        pltpu.async_copy(x_h.at[srcv[0]], rows[0], semG.at[0]).wait()
        chunk_compute(0)
    plsc.subcore_barrier()
    _write_out(c, s, acc, out_h)


@functools.cache
def _build_sc_kernels():
    mesh = plsc.VectorSubcoreMesh(
        core_axis_name="c", subcore_axis_name="s",
        num_cores=NC, num_subcores=NS)
    idx_bufs = [pltpu.VMEM((CK,), jnp.int32) for _ in range(2 * NBUF)]
    f32_bufs = [pltpu.VMEM((CK,), jnp.float32) for _ in range(NBUF)]
    sc_l1 = pl.kernel(
        _sc_layer1,
        out_type=jax.ShapeDtypeStruct((NC, N, H), jnp.float32),
        mesh=mesh,
        scratch_types=(
            idx_bufs + f32_bufs
            + [pltpu.VMEM((CK,), jnp.float32) for _ in range(NBUF)]
            + [
                pltpu.VMEM((CK, H), jnp.float32),
                pltpu.VMEM((3, H), jnp.float32),
                pltpu.VMEM_SHARED((N, H), jnp.float32),
                pltpu.SemaphoreType.DMA((NBUF,)),
                pltpu.SemaphoreType.DMA((NBUF,)),
            ]
        ),
    )
    sc_l2 = pl.kernel(
        _sc_layer2,
        out_type=jax.ShapeDtypeStruct((NC, N, H), jnp.float32),
        mesh=mesh,
        scratch_types=(
            [pltpu.VMEM((CK,), jnp.int32) for _ in range(2 * NBUF2)]
            + [pltpu.VMEM((CK,), jnp.float32) for _ in range(NBUF2)]
            + [pltpu.VMEM((CK, H), jnp.float32) for _ in range(NBUF2)]
            + [
                pltpu.VMEM((2, H), jnp.float32),
                pltpu.VMEM_SHARED((N, H), jnp.float32),
                pltpu.SemaphoreType.DMA((NBUF2,)),
                pltpu.SemaphoreType.DMA((NBUF2,)),
            ]
        ),
    )
    return sc_l1, sc_l2


def _bn_rows(t, g, be):
    m = jnp.mean(t, axis=0, keepdims=True)
    v = jnp.mean((t - m) ** 2, axis=0, keepdims=True)
    return g * (t - m) * lax.rsqrt(v + 1e-5) + be


def _tc_dense1(nw_ref, agg_ref, w0_ref, b0_ref, w1_ref, b1_ref,
               g1_ref, be1_ref, w2_ref, b2_ref, out_ref):
    x = nw_ref[...] * w0_ref[...] + b0_ref[...]
    h = x + agg_ref[0] + agg_ref[1]
    t = jnp.dot(h, w1_ref[...], preferred_element_type=jnp.float32) + b1_ref[...]
    t = jnp.maximum(t, 0.0)
    tn = _bn_rows(t, g1_ref[...], be1_ref[...])
    y = jnp.dot(tn, w2_ref[...], preferred_element_type=jnp.float32) + b2_ref[...]
    out_ref[...] = jnp.maximum(y, 0.0)


def _tc_dense2(x_ref, agg_ref, batch_ref, w1_ref, b1_ref, g1_ref, be1_ref,
               w2_ref, b2_ref, mw1_ref, mb1_ref, mg_ref, mbe_ref,
               mw2_ref, mb2_ref, out_ref):
    h = x_ref[...] + agg_ref[0] + agg_ref[1]
    t = jnp.dot(h, w1_ref[...], preferred_element_type=jnp.float32) + b1_ref[...]
    t = jnp.maximum(t, 0.0)
    tn = _bn_rows(t, g1_ref[...], be1_ref[...])
    y = jnp.dot(tn, w2_ref[...], preferred_element_type=jnp.float32) + b2_ref[...]
    x3 = jnp.maximum(y, 0.0)
    oh = (batch_ref[...] == lax.broadcasted_iota(jnp.int32, (1, G), 1))
    pooled = lax.dot_general(oh.astype(jnp.float32), x3,
                             (((0,), (0,)), ((), ())),
                             preferred_element_type=jnp.float32)
    hh = jnp.dot(pooled, mw1_ref[...], preferred_element_type=jnp.float32)
    hh = jnp.maximum(hh + mb1_ref[...], 0.0)
    hn = _bn_rows(hh, mg_ref[...], mbe_ref[...])
    logits = jnp.dot(hn, mw2_ref[...], preferred_element_type=jnp.float32)
    logits = logits + mb2_ref[...]
    z = logits - jnp.max(logits, axis=1, keepdims=True)
    out_ref[...] = z - jnp.log(jnp.sum(jnp.exp(z), axis=1, keepdims=True))


_tc1 = pl.pallas_call(
    _tc_dense1, out_shape=jax.ShapeDtypeStruct((N, H), jnp.float32))
_tc2 = pl.pallas_call(
    _tc_dense2, out_shape=jax.ShapeDtypeStruct((G, OUT), jnp.float32))


def kernel(edge_index, edge_weight, node_weight, batch,
           node_W, node_b, edge_W, edge_b,
           c1_W1, c1_b1, c1_g, c1_be, c1_W2, c1_b2,
           c2_W1, c2_b1, c2_g, c2_be, c2_W2, c2_b2,
           m_W1, m_b1, m_g, m_be, m_W2, m_b2):
    sc_l1, sc_l2 = _build_sc_kernels()
    src = edge_index[0]
    dst = edge_index[1]
    coef1 = jnp.stack([node_W[0], edge_W[0], node_b + edge_b])
    coef2 = jnp.stack([edge_W[0], edge_b])
    zeros = jnp.zeros((RPT, H), jnp.float32)

    agg1 = sc_l1(src, dst, edge_weight, node_weight, coef1, zeros)
    x2 = _tc1(node_weight.reshape(N, 1), agg1,
              node_W, node_b.reshape(1, H),
              c1_W1, c1_b1.reshape(1, H), c1_g.reshape(1, H),
              c1_be.reshape(1, H), c1_W2, c1_b2.reshape(1, H))
    agg2 = sc_l2(src, dst, edge_weight, x2, coef2, zeros)
    out = _tc2(x2, agg2, batch.reshape(N, 1),
               c2_W1, c2_b1.reshape(1, H), c2_g.reshape(1, H),
               c2_be.reshape(1, H), c2_W2, c2_b2.reshape(1, H),
               m_W1, m_b1.reshape(1, H), m_g.reshape(1, H),
               m_be.reshape(1, H), m_W2, m_b2.reshape(1, OUT))
    return out


# speedup vs baseline: 2.8636x; 1.5029x over previous
import functools

import jax
import jax.numpy as jnp
from jax import lax
from jax.experimental import pallas as pl
from jax.experimental.pallas import tpu as pltpu
from jax.experimental.pallas import tpu_sc as plsc

H = 128
N = 10000
E = 320000
G = 64
OUT = 5

NC = 2
NS = 16
L = 16
NW = NC * NS
EPW = E // NW
CK = 80
NCHUNK = EPW // CK
NBUF = 4
NSUP = NCHUNK // NBUF
NTAIL = NCHUNK - NSUP * NBUF
SCAT_BYTES = CK * H * 4
RPT = 624
TAIL = N - NS * RPT

_NV = H // L

_GD = lax.GatherDimensionNumbers(
    offset_dims=(), collapsed_slice_dims=(0,), start_index_map=(0,))


def _bcast_lane(vec, j):
    idx = jnp.full((L, 1), j, jnp.int32)
    return lax.gather(vec, idx, _GD, (1,),
                      mode=lax.GatherScatterMode.PROMISE_IN_BOUNDS)


def _zero_acc(s, zero_h, acc):
    pltpu.sync_copy(zero_h.at[pl.ds(0, RPT)], acc.at[pl.ds(s * RPT, RPT)])

    @pl.when(s == 0)
    def _zero_tail():
        pltpu.sync_copy(zero_h.at[pl.ds(0, TAIL)], acc.at[pl.ds(NS * RPT, TAIL)])


def _write_out(c, s, acc, out_h):
    pltpu.sync_copy(acc.at[pl.ds(s * RPT, RPT)],
                    out_h.at[c, pl.ds(s * RPT, RPT)])

    @pl.when(s == 0)
    def _write_tail():
        pltpu.sync_copy(acc.at[pl.ds(NS * RPT, TAIL)],
                        out_h.at[c, pl.ds(NS * RPT, TAIL)])


def _sc_layer1(src_h, dst_h, ew_h, nw_h, coef_h, zero_h, out_h, *scr):
    srcv = scr[0:NBUF]
    dstv = scr[NBUF:2 * NBUF]
    ewv = scr[2 * NBUF:3 * NBUF]
    nwg = scr[3 * NBUF:4 * NBUF]
    mbuf = scr[4 * NBUF:5 * NBUF]
    coefv, acc, semL, semG, semS = scr[5 * NBUF:]
    c = lax.axis_index("c")
    s = lax.axis_index("s")
    wid = c * NS + s
    pltpu.sync_copy(coef_h, coefv)
    _zero_acc(s, zero_h, acc)
    plsc.subcore_barrier()
    w0 = [coefv[0, pl.ds(L * v, L)] for v in range(_NV)]
    we = [coefv[1, pl.ds(L * v, L)] for v in range(_NV)]
    cc = [coefv[2, pl.ds(L * v, L)] for v in range(_NV)]

    def chunk_compute(b):
        def group(q, cr):
            b16 = q * L
            nw16 = nwg[b][pl.ds(b16, L)]
            ew16 = ewv[b][pl.ds(b16, L)]

            def edge(j, cj):
                nwb = _bcast_lane(nw16, j)
                ewb = _bcast_lane(ew16, j)
                for v in range(_NV):
                    t = nwb * w0[v] + (ewb * we[v] + cc[v])
                    mbuf[b][b16 + j, pl.ds(L * v, L)] = jnp.maximum(t, 0.0)
                return cj

            lax.fori_loop(0, L, edge, 0)
            return cr

        lax.fori_loop(0, CK // L, group, 0)

    def superiter(g, carry):
        @pl.when(g > 0)
        def _drain_prev():
            for b in range(NBUF):
                pltpu.make_async_copy(mbuf[b], acc.at[dstv[b]], semS.at[b]).wait()

        base0 = wid * EPW + g * NBUF * CK
        lh = []
        for b in range(NBUF):
            base = base0 + b * CK
            lh.append((
                pltpu.async_copy(src_h.at[pl.ds(base, CK)], srcv[b], semL.at[b]),
                pltpu.async_copy(dst_h.at[pl.ds(base, CK)], dstv[b], semL.at[b]),
                pltpu.async_copy(ew_h.at[pl.ds(base, CK)], ewv[b], semL.at[b]),
            ))
        gh = []
        for b in range(NBUF):
            for h in lh[b]:
                h.wait()
            gh.append(pltpu.async_copy(nw_h.at[srcv[b]], nwg[b], semG.at[b]))
        for b in range(NBUF):
            gh[b].wait()
            chunk_compute(b)
            pltpu.async_copy(mbuf[b], acc.at[dstv[b]], semS.at[b], add=True)
        return carry

    lax.fori_loop(0, NSUP, superiter, 0)
    for b in range(NBUF):
        pltpu.make_async_copy(mbuf[b], acc.at[dstv[b]], semS.at[b]).wait()
    for t in range(NTAIL):
        base = wid * EPW + (NSUP * NBUF + t) * CK
        pltpu.sync_copy(src_h.at[pl.ds(base, CK)], srcv[0])
        pltpu.sync_copy(dst_h.at[pl.ds(base, CK)], dstv[0])
        pltpu.sync_copy(ew_h.at[pl.ds(base, CK)], ewv[0])
        pltpu.async_copy(nw_h.at[srcv[0]], nwg[0], semG.at[0]).wait()
        chunk_compute(0)
        pltpu.sync_copy(mbuf[0], acc.at[dstv[0]], add=True)
    plsc.subcore_barrier()
    _write_out(c, s, acc, out_h)


def _sc_layer2(src_h, dst_h, ew_h, x_h, coef_h, zero_h, out_h, *scr):
    srcv = scr[0:NBUF]
    dstv = scr[NBUF:2 * NBUF]
    ewv = scr[2 * NBUF:3 * NBUF]
    rows = scr[3 * NBUF:4 * NBUF]
    coefv, acc, semL, semG, semS = scr[4 * NBUF:]
    c = lax.axis_index("c")
    s = lax.axis_index("s")
    wid = c * NS + s
    pltpu.sync_copy(coef_h, coefv)
    _zero_acc(s, zero_h, acc)
    plsc.subcore_barrier()
    we = [coefv[0, pl.ds(L * v, L)] for v in range(_NV)]
    eb = [coefv[1, pl.ds(L * v, L)] for v in range(_NV)]

    def chunk_compute(b):
        def group(q, cr):
            b16 = q * L
            ew16 = ewv[b][pl.ds(b16, L)]

            def edge(j, cj):
                ewb = _bcast_lane(ew16, j)
                for v in range(_NV):
                    sl = pl.ds(L * v, L)
                    t = rows[b][b16 + j, sl] + (ewb * we[v] + eb[v])
                    rows[b][b16 + j, sl] = jnp.maximum(t, 0.0)
                return cj

            lax.fori_loop(0, L, edge, 0)
            return cr

        lax.fori_loop(0, CK // L, group, 0)

    def superiter(g, carry):
        @pl.when(g > 0)
        def _drain_prev():
            for b in range(NBUF):
                pltpu.make_async_copy(rows[b], acc.at[dstv[b]], semS.at[b]).wait()

        base0 = wid * EPW + g * NBUF * CK
        lh = []
        for b in range(NBUF):
            base = base0 + b * CK
            lh.append((
                pltpu.async_copy(src_h.at[pl.ds(base, CK)], srcv[b], semL.at[b]),
                pltpu.async_copy(dst_h.at[pl.ds(base, CK)], dstv[b], semL.at[b]),
                pltpu.async_copy(ew_h.at[pl.ds(base, CK)], ewv[b], semL.at[b]),
            ))
        gh = []
        for b in range(NBUF):
            for h in lh[b]:
                h.wait()
            gh.append(pltpu.async_copy(x_h.at[srcv[b]], rows[b], semG.at[b]))
        for b in range(NBUF):
            gh[b].wait()
            chunk_compute(b)
            pltpu.async_copy(rows[b], acc.at[dstv[b]], semS.at[b], add=True)
        return carry

    lax.fori_loop(0, NSUP, superiter, 0)
    for b in range(NBUF):
        pltpu.make_async_copy(rows[b], acc.at[dstv[b]], semS.at[b]).wait()
    for t in range(NTAIL):
        base = wid * EPW + (NSUP * NBUF + t) * CK
        pltpu.sync_copy(src_h.at[pl.ds(base, CK)], srcv[0])
        pltpu.sync_copy(dst_h.at[pl.ds(base, CK)], dstv[0])
        pltpu.sync_copy(ew_h.at[pl.ds(base, CK)], ewv[0])
        pltpu.async_copy(x_h.at[srcv[0]], rows[0], semG.at[0]).wait()
        chunk_compute(0)
        pltpu.sync_copy(rows[0], acc.at[dstv[0]], add=True)
    plsc.subcore_barrier()
    _write_out(c, s, acc, out_h)


@functools.cache
def _build_sc_kernels():
    mesh = plsc.VectorSubcoreMesh(
        core_axis_name="c", subcore_axis_name="s",
        num_cores=NC, num_subcores=NS)
    def ring_scratch(n_f32_rings, coef_rows):
        return (
            [pltpu.VMEM((CK,), jnp.int32) for _ in range(2 * NBUF)]
            + [pltpu.VMEM((CK,), jnp.float32)
               for _ in range(n_f32_rings * NBUF)]
            + [pltpu.VMEM((CK, H), jnp.float32) for _ in range(NBUF)]
            + [
                pltpu.VMEM((coef_rows, H), jnp.float32),
                pltpu.VMEM_SHARED((N, H), jnp.float32),
                pltpu.SemaphoreType.DMA((NBUF,)),
                pltpu.SemaphoreType.DMA((NBUF,)),
                pltpu.SemaphoreType.DMA((NBUF,)),
            ]
        )

    sc_l1 = pl.kernel(
        _sc_layer1,
        out_type=jax.ShapeDtypeStruct((NC, N, H), jnp.float32),
        mesh=mesh,
        scratch_types=ring_scratch(2, 3),
    )
    sc_l2 = pl.kernel(
        _sc_layer2,
        out_type=jax.ShapeDtypeStruct((NC, N, H), jnp.float32),
        mesh=mesh,
        scratch_types=ring_scratch(1, 2),
    )
    return sc_l1, sc_l2


def _bn_rows(t, g, be):
    m = jnp.mean(t, axis=0, keepdims=True)
    v = jnp.mean((t - m) ** 2, axis=0, keepdims=True)
    return g * (t - m) * lax.rsqrt(v + 1e-5) + be


def _tc_dense1(nw_ref, agg_ref, w0_ref, b0_ref, w1_ref, b1_ref,
               g1_ref, be1_ref, w2_ref, b2_ref, out_ref):
    x = nw_ref[...] * w0_ref[...] + b0_ref[...]
    h = x + agg_ref[0] + agg_ref[1]
    t = jnp.dot(h, w1_ref[...], preferred_element_type=jnp.float32) + b1_ref[...]
    t = jnp.maximum(t, 0.0)
    tn = _bn_rows(t, g1_ref[...], be1_ref[...])
    y = jnp.dot(tn, w2_ref[...], preferred_element_type=jnp.float32) + b2_ref[...]
    out_ref[...] = jnp.maximum(y, 0.0)


def _tc_dense2(x_ref, agg_ref, batch_ref, w1_ref, b1_ref, g1_ref, be1_ref,
               w2_ref, b2_ref, mw1_ref, mb1_ref, mg_ref, mbe_ref,
               mw2_ref, mb2_ref, out_ref):
    h = x_ref[...] + agg_ref[0] + agg_ref[1]
    t = jnp.dot(h, w1_ref[...], preferred_element_type=jnp.float32) + b1_ref[...]
    t = jnp.maximum(t, 0.0)
    tn = _bn_rows(t, g1_ref[...], be1_ref[...])
    y = jnp.dot(tn, w2_ref[...], preferred_element_type=jnp.float32) + b2_ref[...]
    x3 = jnp.maximum(y, 0.0)
    oh = (batch_ref[...] == lax.broadcasted_iota(jnp.int32, (1, G), 1))
    pooled = lax.dot_general(oh.astype(jnp.float32), x3,
                             (((0,), (0,)), ((), ())),
                             preferred_element_type=jnp.float32)
    hh = jnp.dot(pooled, mw1_ref[...], preferred_element_type=jnp.float32)
    hh = jnp.maximum(hh + mb1_ref[...], 0.0)
    hn = _bn_rows(hh, mg_ref[...], mbe_ref[...])
    logits = jnp.dot(hn, mw2_ref[...], preferred_element_type=jnp.float32)
    logits = logits + mb2_ref[...]
    z = logits - jnp.max(logits, axis=1, keepdims=True)
    out_ref[...] = z - jnp.log(jnp.sum(jnp.exp(z), axis=1, keepdims=True))


_tc1 = pl.pallas_call(
    _tc_dense1, out_shape=jax.ShapeDtypeStruct((N, H), jnp.float32))
_tc2 = pl.pallas_call(
    _tc_dense2, out_shape=jax.ShapeDtypeStruct((G, OUT), jnp.float32))


def kernel(edge_index, edge_weight, node_weight, batch,
           node_W, node_b, edge_W, edge_b,
           c1_W1, c1_b1, c1_g, c1_be, c1_W2, c1_b2,
           c2_W1, c2_b1, c2_g, c2_be, c2_W2, c2_b2,
           m_W1, m_b1, m_g, m_be, m_W2, m_b2):
    sc_l1, sc_l2 = _build_sc_kernels()
    src = edge_index[0]
    dst = edge_index[1]
    coef1 = jnp.stack([node_W[0], edge_W[0], node_b + edge_b])
    coef2 = jnp.stack([edge_W[0], edge_b])
    zeros = jnp.zeros((RPT, H), jnp.float32)

    agg1 = sc_l1(src, dst, edge_weight, node_weight, coef1, zeros)
    x2 = _tc1(node_weight.reshape(N, 1), agg1,
              node_W, node_b.reshape(1, H),
              c1_W1, c1_b1.reshape(1, H), c1_g.reshape(1, H),
              c1_be.reshape(1, H), c1_W2, c1_b2.reshape(1, H))
    agg2 = sc_l2(src, dst, edge_weight, x2, coef2, zeros)
    out = _tc2(x2, agg2, batch.reshape(N, 1),
               c2_W1, c2_b1.reshape(1, H), c2_g.reshape(1, H),
               c2_be.reshape(1, H), c2_W2, c2_b2.reshape(1, H),
               m_W1, m_b1.reshape(1, H), m_g.reshape(1, H),
               m_be.reshape(1, H), m_W2, m_b2.reshape(1, OUT))
    return out
